# speedup vs baseline: 3.2906x; 3.2906x over previous
import functools

import jax
import jax.numpy as jnp
from jax.experimental import pallas as pl
from jax.experimental.pallas import tpu as pltpu

E = 8
TOPK = 2
H = 768
FF = 768
M = 2048
TM = 256
NT = M // TM


def _router_body(x_ref, gw_ref, logits_ref, wdense_ref):
    x = x_ref[...]
    gw = gw_ref[...]
    logits = jax.lax.dot_general(
        x, gw, (((1,), (1,)), ((), ())),
        preferred_element_type=jnp.float32)
    logits_ref[...] = logits

    neg = jnp.float32(-jnp.inf)
    m1 = jnp.full((M, 1), neg, dtype=jnp.float32)
    a1 = jnp.zeros((M, 1), dtype=jnp.int32)
    for e in range(E):
        le = logits[:, e:e + 1]
        take = le > m1
        m1 = jnp.where(take, le, m1)
        a1 = jnp.where(take, jnp.int32(e), a1)
    m2 = jnp.full((M, 1), neg, dtype=jnp.float32)
    a2 = jnp.zeros((M, 1), dtype=jnp.int32)
    for e in range(E):
        le = logits[:, e:e + 1]
        take = (le > m2) & (a1 != jnp.int32(e))
        m2 = jnp.where(take, le, m2)
        a2 = jnp.where(take, jnp.int32(e), a2)

    w1 = 1.0 / (1.0 + jnp.exp(m2 - m1))
    w2 = 1.0 - w1
    lane = jax.lax.broadcasted_iota(jnp.int32, (M, E), 1)
    wdense_ref[...] = (jnp.where(lane == a1, w1, 0.0)
                       + jnp.where(lane == a2, w2, 0.0))


def _moe_body(x_ref, wg_ref, wu_ref, wd_ref, wdense_ref, out_ref):
    e = pl.program_id(0)
    t = pl.program_id(1)
    xt = x_ref[...]
    wg = wg_ref[0]
    wu = wu_ref[0]
    wd = wd_ref[0]

    g = jax.lax.dot_general(xt, wg, (((1,), (1,)), ((), ())),
                            preferred_element_type=jnp.float32)
    u = jax.lax.dot_general(xt, wu, (((1,), (1,)), ((), ())),
                            preferred_element_type=jnp.float32)
    lane = jax.lax.broadcasted_iota(jnp.int32, (TM, E), 1)
    wcol = jnp.sum(jnp.where(lane == e, wdense_ref[...], 0.0), axis=1,
                   keepdims=True)
    h = (g / (1.0 + jnp.exp(-g))) * u * wcol
    y = jax.lax.dot_general(h.astype(jnp.bfloat16), wd,
                            (((1,), (1,)), ((), ())),
                            preferred_element_type=jnp.float32)
    sl = pl.ds(t * TM, TM)

    @pl.when(e == 0)
    def _():
        out_ref[sl, :] = y

    @pl.when(e != 0)
    def _():
        out_ref[sl, :] = out_ref[sl, :] + y


@jax.jit
def kernel(hidden_states, gate_w, gate_proj_w, up_proj_w, down_proj_w):
    B_, S_, H_ = hidden_states.shape
    x = hidden_states.reshape(M, H)

    logits, wdense = pl.pallas_call(
        _router_body,
        out_shape=(
            jax.ShapeDtypeStruct((M, E), jnp.float32),
            jax.ShapeDtypeStruct((M, E), jnp.float32),
        ),
        in_specs=[
            pl.BlockSpec((M, H), lambda: (0, 0)),
            pl.BlockSpec((E, H), lambda: (0, 0)),
        ],
        out_specs=(
            pl.BlockSpec((M, E), lambda: (0, 0)),
            pl.BlockSpec((M, E), lambda: (0, 0)),
        ),
    )(x, gate_w)

    x_bf = x.astype(jnp.bfloat16)
    wg_bf = gate_proj_w.astype(jnp.bfloat16)
    wu_bf = up_proj_w.astype(jnp.bfloat16)
    wd_bf = down_proj_w.astype(jnp.bfloat16)

    out = pl.pallas_call(
        _moe_body,
        grid=(E, NT),
        out_shape=jax.ShapeDtypeStruct((M, H), jnp.float32),
        in_specs=[
            pl.BlockSpec((TM, H), lambda e, t: (t, 0)),
            pl.BlockSpec((1, FF, H), lambda e, t: (e, 0, 0)),
            pl.BlockSpec((1, FF, H), lambda e, t: (e, 0, 0)),
            pl.BlockSpec((1, H, FF), lambda e, t: (e, 0, 0)),
            pl.BlockSpec((TM, E), lambda e, t: (t, 0)),
        ],
        out_specs=pl.BlockSpec((M, H), lambda e, t: (0, 0)),
        compiler_params=pltpu.CompilerParams(
            dimension_semantics=("arbitrary", "arbitrary")),
    )(x_bf, wg_bf, wu_bf, wd_bf, wdense)

    return out.reshape(B_, S_, H_), logits


# speedup vs baseline: 4.8805x; 1.4832x over previous
import jax
import jax.numpy as jnp
from jax.experimental import pallas as pl
from jax.experimental.pallas import tpu as pltpu

E = 8
TOPK = 2
H = 768
FF = 768
M = 2048
TM = 512
NT = M // TM


def _top2_wdense(logits):
    lane = jax.lax.broadcasted_iota(jnp.int32, (M, E), 1)
    m1 = jnp.max(logits, axis=1, keepdims=True)
    a1 = jnp.min(jnp.where(logits == m1, lane, E), axis=1, keepdims=True)
    l2 = jnp.where(lane == a1, -jnp.inf, logits)
    m2 = jnp.max(l2, axis=1, keepdims=True)
    a2 = jnp.min(jnp.where(l2 == m2, lane, E), axis=1, keepdims=True)
    w1 = 1.0 / (1.0 + jnp.exp(m2 - m1))
    w2 = 1.0 - w1
    return jnp.where(lane == a1, w1, 0.0) + jnp.where(lane == a2, w2, 0.0)


def _body(x_ref, gw_ref, wg_ref, wu_ref, wd_ref, out_ref, logits_ref,
          wdense_ref, wgb_ref, wub_ref, wdb_ref):
    e = pl.program_id(0)
    t = pl.program_id(1)

    @pl.when((e == 0) & (t == 0))
    def _():
        logits = jax.lax.dot_general(
            x_ref[...], gw_ref[...], (((1,), (1,)), ((), ())),
            preferred_element_type=jnp.float32)
        logits_ref[...] = logits
        wdense_ref[...] = _top2_wdense(logits)

    @pl.when(t == 0)
    def _():
        wgb_ref[...] = wg_ref[0].astype(jnp.bfloat16)
        wub_ref[...] = wu_ref[0].astype(jnp.bfloat16)
        wdb_ref[...] = wd_ref[0].astype(jnp.bfloat16)

    sl = pl.ds(t * TM, TM)
    xt = x_ref[sl, :].astype(jnp.bfloat16)
    g = jax.lax.dot_general(xt, wgb_ref[...], (((1,), (1,)), ((), ())),
                            preferred_element_type=jnp.float32)
    u = jax.lax.dot_general(xt, wub_ref[...], (((1,), (1,)), ((), ())),
                            preferred_element_type=jnp.float32)
    lane = jax.lax.broadcasted_iota(jnp.int32, (TM, E), 1)
    wcol = jnp.sum(jnp.where(lane == e, wdense_ref[sl, :], 0.0), axis=1,
                   keepdims=True)
    h = (g / (1.0 + jnp.exp(-g))) * u * wcol
    y = jax.lax.dot_general(h.astype(jnp.bfloat16), wdb_ref[...],
                            (((1,), (1,)), ((), ())),
                            preferred_element_type=jnp.float32)

    @pl.when(e == 0)
    def _():
        out_ref[sl, :] = y

    @pl.when(e != 0)
    def _():
        out_ref[sl, :] = out_ref[sl, :] + y


@jax.jit
def kernel(hidden_states, gate_w, gate_proj_w, up_proj_w, down_proj_w):
    B_, S_, H_ = hidden_states.shape
    x = hidden_states.reshape(M, H)

    out, logits = pl.pallas_call(
        _body,
        grid=(E, NT),
        out_shape=(
            jax.ShapeDtypeStruct((M, H), jnp.float32),
            jax.ShapeDtypeStruct((M, E), jnp.float32),
        ),
        in_specs=[
            pl.BlockSpec((M, H), lambda e, t: (0, 0)),
            pl.BlockSpec((E, H), lambda e, t: (0, 0)),
            pl.BlockSpec((1, FF, H), lambda e, t: (e, 0, 0)),
            pl.BlockSpec((1, FF, H), lambda e, t: (e, 0, 0)),
            pl.BlockSpec((1, H, FF), lambda e, t: (e, 0, 0)),
        ],
        out_specs=(
            pl.BlockSpec((M, H), lambda e, t: (0, 0)),
            pl.BlockSpec((M, E), lambda e, t: (0, 0)),
        ),
        scratch_shapes=[
            pltpu.VMEM((M, E), jnp.float32),
            pltpu.VMEM((FF, H), jnp.bfloat16),
            pltpu.VMEM((FF, H), jnp.bfloat16),
            pltpu.VMEM((H, FF), jnp.bfloat16),
        ],
        compiler_params=pltpu.CompilerParams(
            dimension_semantics=("arbitrary", "arbitrary")),
    )(x, gate_w, gate_proj_w, up_proj_w, down_proj_w)

    return out.reshape(B_, S_, H_), logits
